# baseline (speedup 1.0000x reference)
import functools

import jax
import jax.numpy as jnp
from jax import lax
from jax.experimental import pallas as pl
from jax.experimental.pallas import tpu as pltpu
from jax.experimental.pallas import tpu_sc as plsc

D_MODEL = 128
MAX_LEN = 32

_NC = 2
_NS = 16
_NW = _NC * _NS
_LANES = 16

_B = 4 * 2048 * 32
_BPW = _B // _NW
_GB = 128
_G = _BPW // _GB
_NBUF = 2


@functools.partial(
    pl.kernel,
    mesh=plsc.VectorSubcoreMesh(core_axis_name="c", subcore_axis_name="s"),
    out_type=jax.ShapeDtypeStruct((_B, D_MODEL), jnp.float32),
    scratch_types=[
        pltpu.VMEM((_G, _GB), jnp.int32),
        pltpu.VMEM((_NBUF, _GB, D_MODEL), jnp.float32),
        pltpu.VMEM((2 * MAX_LEN + 1, D_MODEL), jnp.float32),
    ]
    + [pltpu.SemaphoreType.DMA] * (1 + _NBUF),
)
def _rpe_lookup(off_hbm, emb_hbm, out_hbm, idx_v, stage_v, table_v, sem_in, *so):
    wid = lax.axis_index("s") * _NC + lax.axis_index("c")

    pltpu.async_copy(emb_hbm, table_v, sem_in)
    pltpu.sync_copy(off_hbm.at[wid], idx_v)
    pltpu.make_async_copy(emb_hbm, table_v, sem_in).wait()

    def clip_body(i, carry):
        r = i // (_GB // _LANES)
        c = (i % (_GB // _LANES)) * _LANES
        v = idx_v[r, pl.ds(c, _LANES)]
        v = jnp.minimum(jnp.maximum(v + MAX_LEN, 0), 2 * MAX_LEN)
        idx_v[r, pl.ds(c, _LANES)] = v
        return carry

    lax.fori_loop(0, _G * (_GB // _LANES), clip_body, 0)

    base = wid * _BPW

    def fill(g, j):
        def row_body(k, carry):
            vec = idx_v[g, pl.ds(k * _LANES, _LANES)]
            rb = k * _LANES
            for l in range(_LANES):
                t = vec[l]
                for c in range(D_MODEL // _LANES):
                    stage_v[j, rb + l, pl.ds(c * _LANES, _LANES)] = table_v[
                        t, pl.ds(c * _LANES, _LANES)
                    ]
            return carry

        lax.fori_loop(0, _GB // _LANES, row_body, 0)

    def fire_o(g, j):
        pltpu.async_copy(stage_v.at[j], out_hbm.at[pl.ds(base + g * _GB, _GB)], so[j])

    def wait_o(g, j):
        pltpu.make_async_copy(
            stage_v.at[j], out_hbm.at[pl.ds(base + g * _GB, _GB)], so[j]
        ).wait()

    fill(0, 0)
    fire_o(0, 0)
    fill(1, 1)
    fire_o(1, 1)

    def main_body(p, carry):
        for u in range(_NBUF):
            g = _NBUF + _NBUF * p + u
            wait_o(g - _NBUF, u)
            fill(g, u)
            fire_o(g, u)
        return carry

    lax.fori_loop(0, (_G - _NBUF) // _NBUF, main_body, 0)

    for u in range(_NBUF):
        wait_o(_G - _NBUF + u, u)


def kernel(offset, embedding):
    off = offset.reshape(_NW, _G, _GB).astype(jnp.int32)
    out = _rpe_lookup(off, embedding)
    return out.reshape(offset.shape + (D_MODEL,))


# speedup vs baseline: 2.4671x; 2.4671x over previous
import functools

import jax
import jax.numpy as jnp
from jax import lax
from jax.experimental import pallas as pl
from jax.experimental.pallas import tpu as pltpu
from jax.experimental.pallas import tpu_sc as plsc

D_MODEL = 128
MAX_LEN = 32

_NC = 2
_NS = 16
_NW = _NC * _NS
_LANES = 16

_B = 4 * 2048 * 32
_BPW = _B // _NW
_GB = 128
_G = _BPW // _GB
_NBUF = 4
_VROWS = 2 * MAX_LEN + 1
_SLOT = 72


@functools.partial(
    pl.kernel,
    mesh=plsc.VectorSubcoreMesh(core_axis_name="c", subcore_axis_name="s"),
    out_type=jax.ShapeDtypeStruct((_B, D_MODEL), jnp.float32),
    scratch_types=[
        pltpu.VMEM((_G, _GB), jnp.int32),
        pltpu.VMEM((_NBUF, _GB, D_MODEL), jnp.float32),
        pltpu.VMEM_SHARED((_NS * _SLOT, D_MODEL), jnp.float32),
    ]
    + [pltpu.SemaphoreType.DMA] * (2 * _NBUF),
)
def _rpe_lookup(off_hbm, emb_hbm, out_hbm, idx_v, rows_v, table_sh, *sems):
    sg, so = sems[:_NBUF], sems[_NBUF:]
    sid = lax.axis_index("s")
    wid = sid * _NC + lax.axis_index("c")

    pltpu.sync_copy(emb_hbm, table_sh.at[pl.ds(sid * _SLOT, _VROWS)])
    pltpu.sync_copy(off_hbm.at[wid], idx_v)

    bias = sid * _SLOT

    def clip_body(i, carry):
        r = i // (_GB // _LANES)
        c = (i % (_GB // _LANES)) * _LANES
        v = idx_v[r, pl.ds(c, _LANES)]
        v = jnp.minimum(jnp.maximum(v + MAX_LEN, 0), 2 * MAX_LEN) + bias
        idx_v[r, pl.ds(c, _LANES)] = v
        return carry

    lax.fori_loop(0, _G * (_GB // _LANES), clip_body, 0)
    plsc.subcore_barrier()

    base = wid * _BPW

    def fire_g(g, j):
        pltpu.async_copy(table_sh.at[idx_v.at[g]], rows_v.at[j], sg[j])

    def wait_g(g, j):
        pltpu.make_async_copy(table_sh.at[idx_v.at[g]], rows_v.at[j], sg[j]).wait()

    def fire_o(g, j):
        pltpu.async_copy(rows_v.at[j], out_hbm.at[pl.ds(base + g * _GB, _GB)], so[j])

    def wait_o(g, j):
        pltpu.make_async_copy(
            rows_v.at[j], out_hbm.at[pl.ds(base + g * _GB, _GB)], so[j]
        ).wait()

    fire_g(0, 0)
    fire_g(1, 1)
    wait_g(0, 0)
    fire_o(0, 0)
    fire_g(2, 2)
    wait_g(1, 1)
    fire_o(1, 1)
    fire_g(3, 3)

    def main_body(p, carry):
        for u in range(4):
            g = 2 + 4 * p + u
            j = (2 + u) % 4
            wait_g(g, j)
            fire_o(g, j)
            jn = u
            wait_o(g - 2, jn)
            fire_g(g + 2, jn)
        return carry

    lax.fori_loop(0, (_G - 4) // 4, main_body, 0)

    for g, j in ((_G - 2, 2), (_G - 1, 3)):
        wait_g(g, j)
        fire_o(g, j)
    for u in range(4):
        wait_o(_G - 4 + u, u)


def kernel(offset, embedding):
    off = offset.reshape(_NW, _G, _GB).astype(jnp.int32)
    out = _rpe_lookup(off, embedding)
    return out.reshape(offset.shape + (D_MODEL,))
